# baseline (speedup 1.0000x reference)
import functools

import jax
import jax.numpy as jnp
from jax import lax
from jax.experimental import pallas as pl
from jax.experimental.pallas import tpu as pltpu
from jax.experimental.pallas import tpu_sc as plsc

N_NODES = 10000
N_EDGES = 320000
D = 128

NC = 2
NS = 16
L = 16
NW = NC * NS
EPW = N_EDGES // NW
K = 80
NCH = EPW // K
NP = 10240
RPT = NP // NS

_mesh = plsc.VectorSubcoreMesh(core_axis_name="c", subcore_axis_name="s")


@functools.partial(
    pl.kernel,
    out_type=jax.ShapeDtypeStruct((2 * NW * N_NODES,), jnp.float32),
    mesh=_mesh,
    scratch_types=[
        pltpu.VMEM((EPW,), jnp.int32),
        pltpu.VMEM((EPW,), jnp.int32),
        pltpu.VMEM((N_NODES,), jnp.float32),
        pltpu.VMEM((N_NODES,), jnp.float32),
    ],
    compiler_params=pltpu.CompilerParams(needs_layout_passes=False),
)
def _deg_kernel(src_hbm, dst_hbm, out_hbm, si_all, di_all, degs_v, degd_v):
    c = lax.axis_index("c")
    s = lax.axis_index("s")
    wid = s * NC + c
    base = wid * EPW
    pltpu.sync_copy(src_hbm.at[pl.ds(base, EPW)], si_all)
    pltpu.sync_copy(dst_hbm.at[pl.ds(base, EPW)], di_all)

    zeros16 = jnp.zeros((L,), jnp.float32)
    ones16 = jnp.ones((L,), jnp.float32)

    def zero_body(i, _):
        degs_v[pl.ds(i * L, L)] = zeros16
        degd_v[pl.ds(i * L, L)] = zeros16
        return _

    lax.fori_loop(0, N_NODES // L, zero_body, None)

    def vec(j, _):
        plsc.addupdate_scatter(degs_v, [si_all[pl.ds(j * L, L)]], ones16)
        plsc.addupdate_scatter(degd_v, [di_all[pl.ds(j * L, L)]], ones16)
        return _

    lax.fori_loop(0, EPW // L, vec, None)

    pltpu.sync_copy(degs_v, out_hbm.at[pl.ds(wid * N_NODES, N_NODES)])
    pltpu.sync_copy(
        degd_v, out_hbm.at[pl.ds((NW + wid) * N_NODES, N_NODES)]
    )


def _mm_body(h_ref, w_ref, z_ref):
    z_ref[...] = jnp.dot(
        h_ref[...], w_ref[...], preferred_element_type=jnp.float32
    )


_mm = pl.pallas_call(
    _mm_body,
    out_shape=jax.ShapeDtypeStruct((N_NODES, D), jnp.float32),
)


def _scale_body(z_ref, degp_ref, hs_ref, iscale_ref):
    out_deg = jnp.maximum(jnp.sum(degp_ref[0], axis=0), 1.0)
    in_deg = jnp.maximum(jnp.sum(degp_ref[1], axis=0), 1.0)
    hs_ref[0:N_NODES, :] = z_ref[...] * lax.rsqrt(out_deg)[:, None]
    iscale_ref[...] = lax.rsqrt(in_deg)[:, None]


_scale = pl.pallas_call(
    _scale_body,
    out_shape=(
        jax.ShapeDtypeStruct((NP, D), jnp.float32),
        jax.ShapeDtypeStruct((N_NODES, 1), jnp.float32),
    ),
)


KM = 80
NCHM = EPW // KM
NG = (NCHM - 1) // 2


@functools.partial(
    pl.kernel,
    out_type=jax.ShapeDtypeStruct((2, NP, D), jnp.float32),
    mesh=_mesh,
    scratch_types=[
        pltpu.VMEM((EPW,), jnp.int32),
        pltpu.VMEM((EPW,), jnp.int32),
        pltpu.VMEM((KM, D), jnp.float32),
        pltpu.VMEM((KM, D), jnp.float32),
        pltpu.VMEM_SHARED((NP, D), jnp.float32),
        pltpu.SemaphoreType.DMA,
        pltpu.SemaphoreType.DMA,
        pltpu.SemaphoreType.DMA,
        pltpu.SemaphoreType.DMA,
    ],
    compiler_params=pltpu.CompilerParams(needs_layout_passes=False),
)
def _msg_kernel(hs_hbm, src_hbm, dst_hbm, out_hbm,
                si_all, di_all, rows0, rows1, acc_sh, gs0, gs1, ss0, ss1):
    c = lax.axis_index("c")
    s = lax.axis_index("s")
    wid = s * NC + c

    base = wid * EPW
    pltpu.sync_copy(src_hbm.at[pl.ds(base, EPW)], si_all)
    pltpu.sync_copy(dst_hbm.at[pl.ds(base, EPW)], di_all)

    zeros16 = jnp.zeros((L,), jnp.float32)

    def zrow(i, _):
        for j in range(D // L):
            rows0[i, pl.ds(j * L, L)] = zeros16
        return _

    lax.fori_loop(0, KM, zrow, None)

    row0 = s * RPT

    def zacc(t, _):
        pltpu.sync_copy(rows0, acc_sh.at[pl.ds(row0 + t * KM, KM)])
        return _

    lax.fori_loop(0, RPT // KM, zacc, None)
    plsc.subcore_barrier()

    def g_idx(i):
        return si_all.at[pl.ds(i * KM, KM)]

    def s_idx(i):
        return di_all.at[pl.ds(i * KM, KM)]

    pltpu.async_copy(hs_hbm.at[g_idx(0)], rows0, gs0)
    pltpu.async_copy(hs_hbm.at[g_idx(1)], rows1, gs1)

    def grp(t, _):
        i0 = 2 * t
        i1 = i0 + 1
        pltpu.make_async_copy(hs_hbm.at[g_idx(i0)], rows0, gs0).wait()
        pltpu.async_copy(rows0, acc_sh.at[s_idx(i0)], ss0, add=True)
        pltpu.make_async_copy(hs_hbm.at[g_idx(i1)], rows1, gs1).wait()
        pltpu.async_copy(rows1, acc_sh.at[s_idx(i1)], ss1, add=True)
        n0 = jnp.minimum(i0 + 2, NCHM - 1)
        n1 = jnp.minimum(i1 + 2, NCHM - 1)
        pltpu.make_async_copy(rows0, acc_sh.at[s_idx(i0)], ss0).wait()
        pltpu.async_copy(hs_hbm.at[g_idx(n0)], rows0, gs0)
        pltpu.make_async_copy(rows1, acc_sh.at[s_idx(i1)], ss1).wait()
        pltpu.async_copy(hs_hbm.at[g_idx(n1)], rows1, gs1)
        return _

    lax.fori_loop(0, NG, grp, None)
    last = NCHM - 1
    pltpu.make_async_copy(hs_hbm.at[g_idx(0)], rows0, gs0).wait()
    pltpu.async_copy(rows0, acc_sh.at[s_idx(last)], ss0, add=True)
    pltpu.make_async_copy(hs_hbm.at[g_idx(0)], rows1, gs1).wait()
    pltpu.make_async_copy(rows0, acc_sh.at[s_idx(last)], ss0).wait()

    plsc.subcore_barrier()
    pltpu.sync_copy(
        acc_sh.at[pl.ds(row0, RPT)], out_hbm.at[c, pl.ds(row0, RPT)]
    )


def _dense2_body(p_ref, iscale_ref, gamma_ref, beta_ref, y_ref):
    agg = (p_ref[0, 0:N_NODES, :] + p_ref[1, 0:N_NODES, :]) * iscale_ref[...]
    mean = jnp.mean(agg, axis=0)
    var = jnp.mean(jnp.square(agg - mean), axis=0)
    y_ref[...] = (agg - mean) * lax.rsqrt(var + 1e-5) * gamma_ref[...] + beta_ref[...]


_dense2 = pl.pallas_call(
    _dense2_body,
    out_shape=jax.ShapeDtypeStruct((N_NODES, D), jnp.float32),
)


@jax.jit
def kernel(h, edge_index, W, gamma, beta):
    ei = edge_index.astype(jnp.int32)
    src, dst = ei[0], ei[1]
    z = _mm(h, W)
    degp = _deg_kernel(src, dst).reshape(2, NW, N_NODES)
    hs, iscale = _scale(z, degp)
    parts = _msg_kernel(hs, src, dst)
    return _dense2(parts, iscale, gamma, beta)


# speedup vs baseline: 1.1754x; 1.1754x over previous
import functools

import jax
import jax.numpy as jnp
from jax import lax
from jax.experimental import pallas as pl
from jax.experimental.pallas import tpu as pltpu
from jax.experimental.pallas import tpu_sc as plsc

N_NODES = 10000
N_EDGES = 320000
D = 128

NC = 2
NS = 16
L = 16
NW = NC * NS
EPW = N_EDGES // NW
K = 80
NCH = EPW // K
NP = 10240
RPT = NP // NS

_mesh = plsc.VectorSubcoreMesh(core_axis_name="c", subcore_axis_name="s")


@functools.partial(
    pl.kernel,
    out_type=jax.ShapeDtypeStruct((2 * NW * N_NODES,), jnp.float32),
    mesh=_mesh,
    scratch_types=[
        pltpu.VMEM((EPW,), jnp.int32),
        pltpu.VMEM((EPW,), jnp.int32),
        pltpu.VMEM((N_NODES,), jnp.float32),
        pltpu.VMEM((N_NODES,), jnp.float32),
    ],
    compiler_params=pltpu.CompilerParams(needs_layout_passes=False),
)
def _deg_kernel(src_hbm, dst_hbm, out_hbm, si_all, di_all, degs_v, degd_v):
    c = lax.axis_index("c")
    s = lax.axis_index("s")
    wid = s * NC + c
    base = wid * EPW
    pltpu.sync_copy(src_hbm.at[pl.ds(base, EPW)], si_all)
    pltpu.sync_copy(dst_hbm.at[pl.ds(base, EPW)], di_all)

    zeros16 = jnp.zeros((L,), jnp.float32)
    ones16 = jnp.ones((L,), jnp.float32)

    def zero_body(i, _):
        degs_v[pl.ds(i * L, L)] = zeros16
        degd_v[pl.ds(i * L, L)] = zeros16
        return _

    lax.fori_loop(0, N_NODES // L, zero_body, None)

    def vec(j, _):
        plsc.addupdate_scatter(degs_v, [si_all[pl.ds(j * L, L)]], ones16)
        plsc.addupdate_scatter(degd_v, [di_all[pl.ds(j * L, L)]], ones16)
        return _

    lax.fori_loop(0, EPW // L, vec, None)

    pltpu.sync_copy(degs_v, out_hbm.at[pl.ds(wid * N_NODES, N_NODES)])
    pltpu.sync_copy(
        degd_v, out_hbm.at[pl.ds((NW + wid) * N_NODES, N_NODES)]
    )


def _dense1_body(h_ref, w_ref, degp_ref, hs_ref, iscale_ref):
    out_deg = jnp.maximum(jnp.sum(degp_ref[0], axis=0), 1.0)
    in_deg = jnp.maximum(jnp.sum(degp_ref[1], axis=0), 1.0)
    hsc = h_ref[...] * lax.rsqrt(out_deg)[:, None]
    hs_ref[0:N_NODES, :] = jnp.dot(
        hsc, w_ref[...], preferred_element_type=jnp.float32
    )
    iscale_ref[...] = lax.rsqrt(in_deg)[:, None]


_dense1 = pl.pallas_call(
    _dense1_body,
    out_shape=(
        jax.ShapeDtypeStruct((NP, D), jnp.float32),
        jax.ShapeDtypeStruct((N_NODES, 1), jnp.float32),
    ),
)


KM = 80
NCHM = EPW // KM
NSLOT = 4
NGM = (NCHM - 1) // NSLOT


@functools.partial(
    pl.kernel,
    out_type=jax.ShapeDtypeStruct((2, NP, D), jnp.float32),
    mesh=_mesh,
    scratch_types=(
        [pltpu.VMEM((KM,), jnp.int32) for _ in range(NSLOT)]
        + [pltpu.VMEM((KM,), jnp.int32) for _ in range(NSLOT)]
        + [pltpu.VMEM((KM, D), jnp.float32) for _ in range(NSLOT)]
        + [pltpu.VMEM_SHARED((NP, D), jnp.float32)]
        + [pltpu.SemaphoreType.DMA for _ in range(3 * NSLOT)]
    ),
    compiler_params=pltpu.CompilerParams(needs_layout_passes=False),
)
def _msg_kernel(hs_hbm, src_hbm, dst_hbm, out_hbm, *scr):
    si = scr[0:NSLOT]
    di = scr[NSLOT:2 * NSLOT]
    rows = scr[2 * NSLOT:3 * NSLOT]
    acc_sh = scr[3 * NSLOT]
    gs = scr[3 * NSLOT + 1:3 * NSLOT + 1 + NSLOT]
    ss = scr[3 * NSLOT + 1 + NSLOT:3 * NSLOT + 1 + 2 * NSLOT]
    isem = scr[3 * NSLOT + 1 + 2 * NSLOT:3 * NSLOT + 1 + 3 * NSLOT]

    c = lax.axis_index("c")
    s = lax.axis_index("s")
    wid = s * NC + c
    base = wid * EPW

    zeros16 = jnp.zeros((L,), jnp.float32)

    def zrow(i, _):
        for j in range(D // L):
            rows[0][i, pl.ds(j * L, L)] = zeros16
        return _

    lax.fori_loop(0, KM, zrow, None)

    row0 = s * RPT

    def zacc(t, _):
        pltpu.sync_copy(rows[0], acc_sh.at[pl.ds(row0 + t * KM, KM)])
        return _

    lax.fori_loop(0, RPT // KM, zacc, None)
    plsc.subcore_barrier()

    def src_sl(i):
        return src_hbm.at[pl.ds(base + i * KM, KM)]

    def dst_sl(i):
        return dst_hbm.at[pl.ds(base + i * KM, KM)]

    for k in range(NSLOT):
        pltpu.sync_copy(src_sl(k), si[k])
        pltpu.sync_copy(dst_sl(k), di[k])
        pltpu.async_copy(hs_hbm.at[si[k]], rows[k], gs[k])

    def grp(t, _):
        i0 = NSLOT * t
        for k in range(NSLOT):
            pltpu.make_async_copy(hs_hbm.at[si[k]], rows[k], gs[k]).wait()
            pltpu.async_copy(rows[k], acc_sh.at[di[k]], ss[k], add=True)
        for k in range(NSLOT):
            n = jnp.minimum(i0 + k + NSLOT, NCHM - 1)
            pltpu.make_async_copy(rows[k], acc_sh.at[di[k]], ss[k]).wait()
            pltpu.async_copy(src_sl(n), si[k], isem[k])
            pltpu.async_copy(dst_sl(n), di[k], isem[k])
            pltpu.make_async_copy(src_sl(n), si[k], isem[k]).wait()
            pltpu.make_async_copy(dst_sl(n), di[k], isem[k]).wait()
            pltpu.async_copy(hs_hbm.at[si[k]], rows[k], gs[k])
        return _

    lax.fori_loop(0, NGM, grp, None)
    pltpu.make_async_copy(hs_hbm.at[si[0]], rows[0], gs[0]).wait()
    pltpu.async_copy(rows[0], acc_sh.at[di[0]], ss[0], add=True)
    for k in range(1, NSLOT):
        pltpu.make_async_copy(hs_hbm.at[si[k]], rows[k], gs[k]).wait()
    pltpu.make_async_copy(rows[0], acc_sh.at[di[0]], ss[0]).wait()

    plsc.subcore_barrier()
    pltpu.sync_copy(
        acc_sh.at[pl.ds(row0, RPT)], out_hbm.at[c, pl.ds(row0, RPT)]
    )


def _dense2_body(p_ref, iscale_ref, gamma_ref, beta_ref, y_ref):
    agg = (p_ref[0, 0:N_NODES, :] + p_ref[1, 0:N_NODES, :]) * iscale_ref[...]
    mean = jnp.mean(agg, axis=0)
    var = jnp.mean(jnp.square(agg - mean), axis=0)
    y_ref[...] = (agg - mean) * lax.rsqrt(var + 1e-5) * gamma_ref[...] + beta_ref[...]


_dense2 = pl.pallas_call(
    _dense2_body,
    out_shape=jax.ShapeDtypeStruct((N_NODES, D), jnp.float32),
)


@jax.jit
def kernel(h, edge_index, W, gamma, beta):
    ei = edge_index.astype(jnp.int32)
    src, dst = ei[0], ei[1]
    degp = _deg_kernel(src, dst).reshape(2, NW, N_NODES)
    hs, iscale = _dense1(h, W, degp)
    parts = _msg_kernel(hs, src, dst)
    return _dense2(parts, iscale, gamma, beta)
